# baseline (speedup 1.0000x reference)
import functools

import jax
import jax.numpy as jnp
from jax import lax
from jax.experimental import pallas as pl
from jax.experimental.pallas import tpu as pltpu
from jax.experimental.pallas import tpu_sc as plsc

N_NODES = 10000
N_EDGES = 160000
D_IN = 256
D_OUT = 256
HALF = 128

NC = 2
NS = 16
LANES = 16

CHUNK = 64
N_CHUNKS = 2560
E_PAD = N_CHUNKS * CHUNK
CPT = N_CHUNKS // NS
GRP = 16
NGRP = CPT // GRP
NBUF = 4
ROWS_MAIN = 624


def _scale_chunk(rows_ref, ev_g, gb2, k):
    def srow(gg, carry):
        evvec = ev_g[gb2, k, pl.ds(gg * LANES, LANES)]
        for l in range(LANES):
            e = evvec[l]
            g = gg * LANES + l
            for p in range(HALF // LANES):
                sl = pl.ds(p * LANES, LANES)
                rows_ref[g, sl] = rows_ref[g, sl] * e
        return carry

    lax.fori_loop(0, CHUNK // LANES, srow, 0)


def _sc_agg_body(x2_hbm, gidx_hbm, dst_hbm, ev_hbm, out_hbm,
                 src_g, dst_g, ev_g, rows_a, rows_b, rows_c, rows_d,
                 acc_sh, gsem, ssem, isem):
    c = lax.axis_index("c")
    s = lax.axis_index("s")
    bufs = (rows_a, rows_b, rows_c, rows_d)
    base = pl.multiple_of(s * CPT, 32)

    def issue_group(gb):
        lo = base + gb * GRP
        pltpu.async_copy(gidx_hbm.at[c, pl.ds(lo, GRP)],
                         src_g.at[gb % 2], isem)
        pltpu.async_copy(dst_hbm.at[pl.ds(lo, GRP)], dst_g.at[gb % 2], isem)
        pltpu.async_copy(ev_hbm.at[pl.ds(lo, GRP)], ev_g.at[gb % 2], isem)

    def wait_group(gb):
        lo = base + gb * GRP
        pltpu.make_async_copy(gidx_hbm.at[c, pl.ds(lo, GRP)],
                              src_g.at[gb % 2], isem).wait()
        pltpu.make_async_copy(dst_hbm.at[pl.ds(lo, GRP)],
                              dst_g.at[gb % 2], isem).wait()
        pltpu.make_async_copy(ev_hbm.at[pl.ds(lo, GRP)],
                              ev_g.at[gb % 2], isem).wait()

    issue_group(0)
    zero16 = jnp.zeros((LANES,), jnp.float32)

    def zrow(r, carry):
        for p in range(HALF // LANES):
            rows_a[r, pl.ds(p * LANES, LANES)] = zero16
        return carry

    lax.fori_loop(0, CHUNK, zrow, 0)
    start = pl.multiple_of(s * ROWS_MAIN, 8)
    for k in range(9):
        pltpu.sync_copy(rows_a, acc_sh.at[pl.ds(start + k * CHUNK, CHUNK)])
    pltpu.sync_copy(rows_a.at[pl.ds(0, ROWS_MAIN - 9 * CHUNK)],
                    acc_sh.at[pl.ds(start + 9 * CHUNK, ROWS_MAIN - 9 * CHUNK)])

    @pl.when(s == NS - 1)
    def _():
        pltpu.sync_copy(rows_a.at[pl.ds(0, N_NODES - NS * ROWS_MAIN)],
                        acc_sh.at[pl.ds(NS * ROWS_MAIN, N_NODES - NS * ROWS_MAIN)])

    plsc.subcore_barrier()

    def start_gather(j, buf):
        pltpu.async_copy(x2_hbm.at[src_g.at[(j // GRP) % 2, j % GRP]],
                         buf, gsem)

    def wait_gather(j, buf):
        pltpu.make_async_copy(x2_hbm.at[src_g.at[(j // GRP) % 2, j % GRP]],
                              buf, gsem).wait()

    def start_scatter(j, buf):
        pltpu.async_copy(buf, acc_sh.at[dst_g.at[(j // GRP) % 2, j % GRP]],
                         ssem, add=True)

    def wait_scatter(buf):
        pltpu.make_async_copy(buf, acc_sh.at[dst_g.at[0, 0]], ssem).wait()

    wait_group(0)
    issue_group(1)
    start_gather(0, bufs[0])
    start_gather(1, bufs[1])

    def quad_body(jo, carry):
        for jj in range(NBUF):
            j = jo * NBUF + jj
            buf = bufs[jj]
            nxt = bufs[(jj + 2) % NBUF]

            @pl.when((j % GRP == GRP - 4) & (j + 4 < CPT))
            def _():
                wait_group(j // GRP + 1)

            @pl.when((j % GRP == 2) & (j // GRP >= 1) & (j // GRP + 1 < NGRP))
            def _():
                issue_group(j // GRP + 1)

            @pl.when(j >= 2)
            def _():
                wait_scatter(nxt)

            @pl.when(j + 2 < CPT)
            def _():
                start_gather(j + 2, nxt)

            wait_gather(j, buf)
            _scale_chunk(buf, ev_g, (j // GRP) % 2, j % GRP)
            start_scatter(j, buf)
        return carry

    lax.fori_loop(0, CPT // NBUF, quad_body, 0)

    wait_scatter(rows_a)
    wait_scatter(rows_b)

    plsc.subcore_barrier()

    pltpu.sync_copy(acc_sh.at[pl.ds(start, ROWS_MAIN)],
                    out_hbm.at[c, pl.ds(start, ROWS_MAIN)])

    @pl.when(s == NS - 1)
    def _():
        pltpu.sync_copy(
            acc_sh.at[pl.ds(NS * ROWS_MAIN, N_NODES - NS * ROWS_MAIN)],
            out_hbm.at[c, pl.ds(NS * ROWS_MAIN, N_NODES - NS * ROWS_MAIN)])


_sc_agg = functools.partial(
    pl.kernel,
    out_type=jax.ShapeDtypeStruct((NC, N_NODES, HALF), jnp.float32),
    mesh=plsc.VectorSubcoreMesh(core_axis_name="c", subcore_axis_name="s"),
    scratch_types=[
        pltpu.VMEM((2, GRP, CHUNK), jnp.int32),
        pltpu.VMEM((2, GRP, CHUNK), jnp.int32),
        pltpu.VMEM((2, GRP, CHUNK), jnp.float32),
        pltpu.VMEM((CHUNK, HALF), jnp.float32),
        pltpu.VMEM((CHUNK, HALF), jnp.float32),
        pltpu.VMEM((CHUNK, HALF), jnp.float32),
        pltpu.VMEM((CHUNK, HALF), jnp.float32),
        pltpu.VMEM_SHARED((N_NODES, HALF), jnp.float32),
        pltpu.SemaphoreType.DMA,
        pltpu.SemaphoreType.DMA,
        pltpu.SemaphoreType.DMA,
    ],
)(_sc_agg_body)


def _mm_body(a0_ref, a1_ref, w0_ref, w1_ref, o_ref):
    acc = jnp.dot(a0_ref[...], w0_ref[...],
                  preferred_element_type=jnp.float32,
                  precision=lax.Precision.HIGHEST)
    acc = acc + jnp.dot(a1_ref[...], w1_ref[...],
                        preferred_element_type=jnp.float32,
                        precision=lax.Precision.HIGHEST)
    o_ref[...] = jnp.maximum(acc, 0.0)


M_BLK = 1000


def _mm_relu(agg2, w):
    return pl.pallas_call(
        _mm_body,
        grid=(N_NODES // M_BLK,),
        in_specs=[
            pl.BlockSpec((M_BLK, HALF), lambda i: (i, 0)),
            pl.BlockSpec((M_BLK, HALF), lambda i: (i, 0)),
            pl.BlockSpec((HALF, D_OUT), lambda i: (0, 0)),
            pl.BlockSpec((HALF, D_OUT), lambda i: (0, 0)),
        ],
        out_specs=pl.BlockSpec((M_BLK, D_OUT), lambda i: (i, 0)),
        out_shape=jax.ShapeDtypeStruct((N_NODES, D_OUT), jnp.float32),
    )(agg2[0], agg2[1], w[:HALF], w[HALF:])


def kernel(x, edge_index, edge_values, W):
    x2 = x.reshape(2 * N_NODES, HALF)
    pad = E_PAD - N_EDGES
    src = edge_index[1]
    gidx = jnp.stack([2 * src, 2 * src + 1])
    gidx = jnp.pad(gidx, ((0, 0), (0, pad))).reshape(2, N_CHUNKS, CHUNK)
    dst2 = jnp.pad(edge_index[0], (0, pad)).reshape(N_CHUNKS, CHUNK)
    ev2 = jnp.pad(edge_values, (0, pad)).reshape(N_CHUNKS, CHUNK)
    agg2 = _sc_agg(x2, gidx, dst2, ev2)
    return _mm_relu(agg2, W)


# speedup vs baseline: 1.0345x; 1.0345x over previous
import functools

import jax
import jax.numpy as jnp
from jax import lax
from jax.experimental import pallas as pl
from jax.experimental.pallas import tpu as pltpu
from jax.experimental.pallas import tpu_sc as plsc

N_NODES = 10000
N_EDGES = 160000
D_IN = 256
D_OUT = 256
HALF = 128

NC = 2
NS = 16
LANES = 16

CHUNK = 64
N_CHUNKS = 2560
E_PAD = N_CHUNKS * CHUNK
CPT = N_CHUNKS // NS
GRP = 16
NGRP = CPT // GRP
NBUF = 4
ROWS_MAIN = 624


def _scale_chunk(rows_ref, ev_g, gb2, k):
    @functools.partial(plsc.parallel_loop, 0, CHUNK // LANES, unroll=2)
    def srow(gg):
        evvec = ev_g[gb2, k, pl.ds(gg * LANES, LANES)]
        for l in range(LANES):
            e = evvec[l]
            g = gg * LANES + l
            for p in range(HALF // LANES):
                sl = pl.ds(p * LANES, LANES)
                rows_ref[g, sl] = rows_ref[g, sl] * e


def _sc_agg_body(x2_hbm, gidx_hbm, dst_hbm, ev_hbm, out_hbm,
                 src_g, dst_g, ev_g, rows_a, rows_b, rows_c, rows_d,
                 acc_sh, gsem, ssem, isem):
    c = lax.axis_index("c")
    s = lax.axis_index("s")
    bufs = (rows_a, rows_b, rows_c, rows_d)
    base = pl.multiple_of(s * CPT, 32)

    def issue_group(gb):
        lo = base + gb * GRP
        pltpu.async_copy(gidx_hbm.at[c, pl.ds(lo, GRP)],
                         src_g.at[gb % 2], isem)
        pltpu.async_copy(dst_hbm.at[pl.ds(lo, GRP)], dst_g.at[gb % 2], isem)
        pltpu.async_copy(ev_hbm.at[pl.ds(lo, GRP)], ev_g.at[gb % 2], isem)

    def wait_group(gb):
        lo = base + gb * GRP
        pltpu.make_async_copy(gidx_hbm.at[c, pl.ds(lo, GRP)],
                              src_g.at[gb % 2], isem).wait()
        pltpu.make_async_copy(dst_hbm.at[pl.ds(lo, GRP)],
                              dst_g.at[gb % 2], isem).wait()
        pltpu.make_async_copy(ev_hbm.at[pl.ds(lo, GRP)],
                              ev_g.at[gb % 2], isem).wait()

    issue_group(0)
    zero16 = jnp.zeros((LANES,), jnp.float32)

    def zrow(r, carry):
        for p in range(HALF // LANES):
            rows_a[r, pl.ds(p * LANES, LANES)] = zero16
        return carry

    lax.fori_loop(0, CHUNK, zrow, 0)
    start = pl.multiple_of(s * ROWS_MAIN, 8)
    for k in range(9):
        pltpu.sync_copy(rows_a, acc_sh.at[pl.ds(start + k * CHUNK, CHUNK)])
    pltpu.sync_copy(rows_a.at[pl.ds(0, ROWS_MAIN - 9 * CHUNK)],
                    acc_sh.at[pl.ds(start + 9 * CHUNK, ROWS_MAIN - 9 * CHUNK)])

    @pl.when(s == NS - 1)
    def _():
        pltpu.sync_copy(rows_a.at[pl.ds(0, N_NODES - NS * ROWS_MAIN)],
                        acc_sh.at[pl.ds(NS * ROWS_MAIN, N_NODES - NS * ROWS_MAIN)])

    plsc.subcore_barrier()

    def start_gather(j, buf):
        pltpu.async_copy(x2_hbm.at[src_g.at[(j // GRP) % 2, j % GRP]],
                         buf, gsem)

    def wait_gather(j, buf):
        pltpu.make_async_copy(x2_hbm.at[src_g.at[(j // GRP) % 2, j % GRP]],
                              buf, gsem).wait()

    def start_scatter(j, buf):
        pltpu.async_copy(buf, acc_sh.at[dst_g.at[(j // GRP) % 2, j % GRP]],
                         ssem, add=True)

    def wait_scatter(buf):
        pltpu.make_async_copy(buf, acc_sh.at[dst_g.at[0, 0]], ssem).wait()

    wait_group(0)
    issue_group(1)
    start_gather(0, bufs[0])
    start_gather(1, bufs[1])

    def quad_body(jo, carry):
        for jj in range(NBUF):
            j = jo * NBUF + jj
            buf = bufs[jj]
            nxt = bufs[(jj + 2) % NBUF]

            @pl.when((j % GRP == GRP - 4) & (j + 4 < CPT))
            def _():
                wait_group(j // GRP + 1)

            @pl.when((j % GRP == 2) & (j // GRP >= 1) & (j // GRP + 1 < NGRP))
            def _():
                issue_group(j // GRP + 1)

            @pl.when(j >= 2)
            def _():
                wait_scatter(nxt)

            @pl.when(j + 2 < CPT)
            def _():
                start_gather(j + 2, nxt)

            wait_gather(j, buf)
            _scale_chunk(buf, ev_g, (j // GRP) % 2, j % GRP)
            start_scatter(j, buf)
        return carry

    lax.fori_loop(0, CPT // NBUF, quad_body, 0)

    wait_scatter(rows_a)
    wait_scatter(rows_b)

    plsc.subcore_barrier()

    pltpu.sync_copy(acc_sh.at[pl.ds(start, ROWS_MAIN)],
                    out_hbm.at[c, pl.ds(start, ROWS_MAIN)])

    @pl.when(s == NS - 1)
    def _():
        pltpu.sync_copy(
            acc_sh.at[pl.ds(NS * ROWS_MAIN, N_NODES - NS * ROWS_MAIN)],
            out_hbm.at[c, pl.ds(NS * ROWS_MAIN, N_NODES - NS * ROWS_MAIN)])


_sc_agg = functools.partial(
    pl.kernel,
    out_type=jax.ShapeDtypeStruct((NC, N_NODES, HALF), jnp.float32),
    mesh=plsc.VectorSubcoreMesh(core_axis_name="c", subcore_axis_name="s"),
    scratch_types=[
        pltpu.VMEM((2, GRP, CHUNK), jnp.int32),
        pltpu.VMEM((2, GRP, CHUNK), jnp.int32),
        pltpu.VMEM((2, GRP, CHUNK), jnp.float32),
        pltpu.VMEM((CHUNK, HALF), jnp.float32),
        pltpu.VMEM((CHUNK, HALF), jnp.float32),
        pltpu.VMEM((CHUNK, HALF), jnp.float32),
        pltpu.VMEM((CHUNK, HALF), jnp.float32),
        pltpu.VMEM_SHARED((N_NODES, HALF), jnp.float32),
        pltpu.SemaphoreType.DMA,
        pltpu.SemaphoreType.DMA,
        pltpu.SemaphoreType.DMA,
    ],
)(_sc_agg_body)


def _mm_body(a0_ref, a1_ref, w0_ref, w1_ref, o_ref):
    acc = jnp.dot(a0_ref[...], w0_ref[...],
                  preferred_element_type=jnp.float32,
                  precision=lax.Precision.HIGHEST)
    acc = acc + jnp.dot(a1_ref[...], w1_ref[...],
                        preferred_element_type=jnp.float32,
                        precision=lax.Precision.HIGHEST)
    o_ref[...] = jnp.maximum(acc, 0.0)


M_BLK = 1000


def _mm_relu(agg2, w):
    return pl.pallas_call(
        _mm_body,
        grid=(N_NODES // M_BLK,),
        in_specs=[
            pl.BlockSpec((M_BLK, HALF), lambda i: (i, 0)),
            pl.BlockSpec((M_BLK, HALF), lambda i: (i, 0)),
            pl.BlockSpec((HALF, D_OUT), lambda i: (0, 0)),
            pl.BlockSpec((HALF, D_OUT), lambda i: (0, 0)),
        ],
        out_specs=pl.BlockSpec((M_BLK, D_OUT), lambda i: (i, 0)),
        out_shape=jax.ShapeDtypeStruct((N_NODES, D_OUT), jnp.float32),
    )(agg2[0], agg2[1], w[:HALF], w[HALF:])


def kernel(x, edge_index, edge_values, W):
    x2 = x.reshape(2 * N_NODES, HALF)
    pad = E_PAD - N_EDGES
    src = edge_index[1]
    gidx = jnp.stack([2 * src, 2 * src + 1])
    gidx = jnp.pad(gidx, ((0, 0), (0, pad))).reshape(2, N_CHUNKS, CHUNK)
    dst2 = jnp.pad(edge_index[0], (0, pad)).reshape(N_CHUNKS, CHUNK)
    ev2 = jnp.pad(edge_values, (0, pad)).reshape(N_CHUNKS, CHUNK)
    agg2 = _sc_agg(x2, gidx, dst2, ev2)
    return _mm_relu(agg2, W)
